# baseline (speedup 1.0000x reference)
import jax
import jax.numpy as jnp
from jax import lax
from jax.experimental import pallas as pl
from jax.experimental.pallas import tpu as pltpu
from jax.experimental.pallas import tpu_sc as plsc

NUM_CORES = 2
NUM_SUBCORES = 16
NW = NUM_CORES * NUM_SUBCORES
NBUF = 4


def _emb_body(x_hbm, w_hbm, out_hbm, idx_v,
              b0, b1, b2, b3, g0, g1, g2, g3, s0, s1, s2, s3):
    nct = x_hbm.shape[0]
    ib = out_hbm.shape[0] // 128
    npw = nct // NW
    wid = lax.axis_index("s") * NUM_CORES + lax.axis_index("c")
    base = wid * npw
    pltpu.sync_copy(x_hbm.at[pl.ds(base, npw)], idx_v)

    bufs = (b0, b1, b2, b3)
    gsems = (g0, g1, g2, g3)
    ssems = (s0, s1, s2, s3)

    def out_slice(m):
        j = m // ib
        iblk = m % ib
        return out_hbm.at[pl.ds(iblk * 128, 128), j, pl.ds(0, 64)]

    for b in range(NBUF - 1):
        pltpu.async_copy(w_hbm.at[idx_v.at[b]], bufs[b], gsems[b])

    def body(i, carry):
        t0 = i * NBUF
        for b in range(NBUF):
            t = t0 + b
            pltpu.make_async_copy(w_hbm.at[idx_v.at[t]], bufs[b],
                                  gsems[b]).wait()
            pltpu.async_copy(bufs[b], out_slice(base + t), ssems[b])
            nb = (b + NBUF - 1) % NBUF
            tn = t + NBUF - 1

            @pl.when(tn < npw)
            def _(nb=nb, tn=tn):
                @pl.when(tn >= NBUF)
                def _():
                    pltpu.make_async_copy(
                        bufs[nb], out_slice(base), ssems[nb]).wait()
                pltpu.async_copy(w_hbm.at[idx_v.at[tn]], bufs[nb], gsems[nb])
        return carry

    lax.fori_loop(0, npw // NBUF, body, 0)
    for b in range(NBUF):
        pltpu.make_async_copy(bufs[b], out_slice(base), ssems[b]).wait()


def kernel(x, W):
    B, S = x.shape
    V, D = W.shape
    ib = B // 128
    nct = S * ib
    x_r = (x.astype(jnp.int32) * 2).T.reshape(nct, 128)
    W = jnp.concatenate([W, W], axis=1).reshape(2 * V, D)

    mesh = plsc.VectorSubcoreMesh(core_axis_name="c", subcore_axis_name="s")
    out = pl.kernel(
        _emb_body,
        out_type=jax.ShapeDtypeStruct((B, 32, 128), jnp.float32),
        mesh=mesh,
        scratch_types=(
            [pltpu.VMEM((nct // NW, 128), jnp.int32)]
            + [pltpu.VMEM((128, D), jnp.float32)] * NBUF
            + [pltpu.SemaphoreType.DMA] * (2 * NBUF)
        ),
        compiler_params=pltpu.CompilerParams(use_tc_tiling_on_sc=False),
    )(x_r, W)
    return out[:, :S, :D]


# speedup vs baseline: 1.2129x; 1.2129x over previous
import jax
import jax.numpy as jnp
from jax import lax
from jax.experimental import pallas as pl
from jax.experimental.pallas import tpu as pltpu
from jax.experimental.pallas import tpu_sc as plsc

NUM_CORES = 2
NUM_SUBCORES = 16
NW = NUM_CORES * NUM_SUBCORES
NBUF = 4


def _emb_body(x_hbm, w_hbm, out_hbm, idx_v,
              b0, b1, b2, b3, g0, g1, g2, g3, s0, s1, s2, s3):
    nct = x_hbm.shape[0]
    ib = out_hbm.shape[0] // 128
    npw = nct // NW
    wid = lax.axis_index("s") * NUM_CORES + lax.axis_index("c")
    base = wid * npw
    pltpu.sync_copy(x_hbm.at[pl.ds(base, npw)], idx_v)

    bufs = (b0, b1, b2, b3)
    gsems = (g0, g1, g2, g3)
    ssems = (s0, s1, s2, s3)

    def out_slice(m):
        j = m // ib
        iblk = m % ib
        return out_hbm.at[pl.ds(iblk * 128, 128), j, pl.ds(0, 64)]

    for b in range(NBUF - 1):
        pltpu.async_copy(w_hbm.at[idx_v.at[b]], bufs[b], gsems[b])

    def body(i, carry):
        t0 = i * NBUF
        for b in range(NBUF):
            t = t0 + b
            pltpu.make_async_copy(w_hbm.at[idx_v.at[t]], bufs[b],
                                  gsems[b]).wait()
            pltpu.async_copy(bufs[b], out_slice(base + t), ssems[b])
            nb = (b + NBUF - 1) % NBUF
            tn = t + NBUF - 1

            @pl.when(tn < npw)
            def _(nb=nb, tn=tn):
                @pl.when(tn >= NBUF)
                def _():
                    pltpu.make_async_copy(
                        bufs[nb], out_slice(base), ssems[nb]).wait()
                pltpu.async_copy(w_hbm.at[idx_v.at[tn]], bufs[nb], gsems[nb])
        return carry

    lax.fori_loop(0, npw // NBUF, body, 0)
    for b in range(NBUF):
        pltpu.make_async_copy(bufs[b], out_slice(base), ssems[b]).wait()


def kernel(x, W):
    B, S = x.shape
    V, D = W.shape
    ib = B // 128
    nct = S * ib
    x_r = (x.astype(jnp.int32) * 2).T.reshape(nct, 128)
    W = jnp.pad(W, ((0, 0), (0, D))).reshape(2 * V, D)

    mesh = plsc.VectorSubcoreMesh(core_axis_name="c", subcore_axis_name="s")
    out = pl.kernel(
        _emb_body,
        out_type=jax.ShapeDtypeStruct((B, 32, 128), jnp.float32),
        mesh=mesh,
        scratch_types=(
            [pltpu.VMEM((nct // NW, 128), jnp.int32)]
            + [pltpu.VMEM((128, D), jnp.float32)] * NBUF
            + [pltpu.SemaphoreType.DMA] * (2 * NBUF)
        ),
        compiler_params=pltpu.CompilerParams(use_tc_tiling_on_sc=False),
    )(x_r, W)
    return out[:, :S, :D]


# speedup vs baseline: 1.2149x; 1.0016x over previous
import jax
import jax.numpy as jnp
from jax import lax
from jax.experimental import pallas as pl
from jax.experimental.pallas import tpu as pltpu
from jax.experimental.pallas import tpu_sc as plsc

NUM_CORES = 2
NUM_SUBCORES = 16
NW = NUM_CORES * NUM_SUBCORES
NBUF = 8


def _emb_body(x_hbm, w_hbm, out_hbm, idx_v,
              b0, b1, b2, b3, b4, b5, b6, b7,
              g0, g1, g2, g3, g4, g5, g6, g7,
              s0, s1, s2, s3, s4, s5, s6, s7):
    nct = x_hbm.shape[0]
    ib = out_hbm.shape[0] // 128
    npw = nct // NW
    wid = lax.axis_index("s") * NUM_CORES + lax.axis_index("c")
    base = wid * npw
    pltpu.sync_copy(x_hbm.at[pl.ds(base, npw)], idx_v)

    bufs = (b0, b1, b2, b3, b4, b5, b6, b7)
    gsems = (g0, g1, g2, g3, g4, g5, g6, g7)
    ssems = (s0, s1, s2, s3, s4, s5, s6, s7)

    def out_slice(m):
        j = m // ib
        iblk = m % ib
        return out_hbm.at[pl.ds(iblk * 128, 128), j, pl.ds(0, 64)]

    for b in range(NBUF - 1):
        pltpu.async_copy(w_hbm.at[idx_v.at[b]], bufs[b], gsems[b])

    def body(i, carry):
        t0 = i * NBUF
        for b in range(NBUF):
            t = t0 + b
            pltpu.make_async_copy(w_hbm.at[idx_v.at[t]], bufs[b],
                                  gsems[b]).wait()
            pltpu.async_copy(bufs[b], out_slice(base + t), ssems[b])
            nb = (b + NBUF - 1) % NBUF
            tn = t + NBUF - 1

            @pl.when(tn < npw)
            def _(nb=nb, tn=tn):
                @pl.when(tn >= NBUF)
                def _():
                    pltpu.make_async_copy(
                        bufs[nb], out_slice(base), ssems[nb]).wait()
                pltpu.async_copy(w_hbm.at[idx_v.at[tn]], bufs[nb], gsems[nb])
        return carry

    lax.fori_loop(0, npw // NBUF, body, 0)
    for b in range(NBUF):
        pltpu.make_async_copy(bufs[b], out_slice(base), ssems[b]).wait()


def kernel(x, W):
    B, S = x.shape
    V, D = W.shape
    ib = B // 128
    nct = S * ib
    x_r = (x.astype(jnp.int32) * 2).T.reshape(nct, 128)
    W = jnp.pad(W, ((0, 0), (0, D))).reshape(2 * V, D)

    mesh = plsc.VectorSubcoreMesh(core_axis_name="c", subcore_axis_name="s")
    out = pl.kernel(
        _emb_body,
        out_type=jax.ShapeDtypeStruct((B, 32, 128), jnp.float32),
        mesh=mesh,
        scratch_types=(
            [pltpu.VMEM((nct // NW, 128), jnp.int32)]
            + [pltpu.VMEM((128, D), jnp.float32)] * NBUF
            + [pltpu.SemaphoreType.DMA] * (2 * NBUF)
        ),
        compiler_params=pltpu.CompilerParams(use_tc_tiling_on_sc=False),
    )(x_r, W)
    return out[:, :S, :D]
